# baseline (speedup 1.0000x reference)
import functools

import jax
import jax.numpy as jnp
from jax import lax
from jax.experimental import pallas as pl
from jax.experimental.pallas import tpu as pltpu
from jax.experimental.pallas import tpu_sc as plsc

NC = 2
NS = 16
CH = 128


def _sc_mesh():
    return plsc.VectorSubcoreMesh(core_axis_name="c", subcore_axis_name="s")


def _make_deg_kernel(n_pad, e_pad):
    chunks_per_tile = e_pad // (NS * CH)
    rows_per_tile = n_pad // NS

    @functools.partial(
        pl.kernel,
        mesh=_sc_mesh(),
        out_type=jax.ShapeDtypeStruct((n_pad,), jnp.float32),
        scratch_types=[
            pltpu.VMEM((CH,), jnp.int32),
            pltpu.VMEM((CH,), jnp.float32),
            pltpu.VMEM((rows_per_tile,), jnp.float32),
            pltpu.VMEM_SHARED((n_pad,), jnp.float32),
            pltpu.SemaphoreType.DMA,
        ],
    )
    def deg_kernel(dst_hbm, ones_hbm, out_hbm, dst_v, ones_v, row_v, deg_sh, sem):
        c = lax.axis_index("c")
        s = lax.axis_index("s")
        row0 = s * rows_per_tile

        @pl.when(c == 0)
        def _():
            pltpu.sync_copy(ones_hbm.at[pl.ds(row0, rows_per_tile)], row_v)
            pltpu.sync_copy(row_v, deg_sh.at[pl.ds(row0, rows_per_tile)])
            pltpu.sync_copy(ones_hbm.at[pl.ds(0, CH)], ones_v)

        plsc.subcore_barrier()

        @pl.when(c == 0)
        def _():
            base_chunk = s * chunks_per_tile

            @pl.loop(0, chunks_per_tile)
            def _(j):
                off = (base_chunk + j) * CH
                pltpu.sync_copy(dst_hbm.at[pl.ds(off, CH)], dst_v)
                pltpu.sync_copy(ones_v, deg_sh.at[dst_v], add=True)

        plsc.subcore_barrier()

        @pl.when(c == 0)
        def _():
            pltpu.sync_copy(deg_sh.at[pl.ds(row0, rows_per_tile)], row_v)
            pltpu.sync_copy(row_v, out_hbm.at[pl.ds(row0, rows_per_tile)])

    return deg_kernel


IDX_Q = 8


def _make_agg_kernel(n_pad, e_pad, dim):
    chunks_per_tile = e_pad // (NC * NS * CH)
    assert chunks_per_tile % IDX_Q == 0
    rows_per_tile = n_pad // NS
    n_init_chunks = (rows_per_tile + CH - 1) // CH

    @functools.partial(
        pl.kernel,
        mesh=_sc_mesh(),
        out_type=jax.ShapeDtypeStruct((NC, n_pad, dim), jnp.float32),
        scratch_types=[
            pltpu.VMEM((2, CH), jnp.int32),
            pltpu.VMEM((CH, dim), jnp.float32),
            pltpu.VMEM_SHARED((n_pad, dim), jnp.float32),
            pltpu.SemaphoreType.DMA,
        ],
    )
    def agg_kernel(h_hbm, zeros_hbm, idx_hbm, out_hbm,
                   idx_v, rows_v, y_sh, sem):
        c = lax.axis_index("c")
        s = lax.axis_index("s")
        row0 = s * rows_per_tile

        for k in range(n_init_chunks):
            r = row0 + k * CH
            m = min(CH, rows_per_tile - k * CH)

            @pl.when(c == 0)
            def _():
                pltpu.sync_copy(h_hbm.at[pl.ds(r, m)],
                                rows_v.at[pl.ds(0, m)])

            @pl.when(c != 0)
            def _():
                pltpu.sync_copy(zeros_hbm.at[pl.ds(r, m)],
                                rows_v.at[pl.ds(0, m)])

            pltpu.sync_copy(rows_v.at[pl.ds(0, m)], y_sh.at[pl.ds(r, m)])

        plsc.subcore_barrier()

        base_chunk = (c * NS + s) * chunks_per_tile

        @pl.loop(0, chunks_per_tile)
        def _(j):
            pltpu.sync_copy(idx_hbm.at[base_chunk + j], idx_v)
            pltpu.async_copy(h_hbm.at[idx_v.at[0]], rows_v, sem).wait()
            pltpu.sync_copy(rows_v, y_sh.at[idx_v.at[1]], add=True)

        plsc.subcore_barrier()

        for k in range(n_init_chunks):
            r = row0 + k * CH
            m = min(CH, rows_per_tile - k * CH)
            pltpu.sync_copy(y_sh.at[pl.ds(r, m)], rows_v.at[pl.ds(0, m)])
            pltpu.sync_copy(rows_v.at[pl.ds(0, m)],
                            out_hbm.at[c, pl.ds(r, m), :])

    return agg_kernel


def _mm_scale_body(n, fts_ref, w_ref, deg_ref, out_ref):
    dinv = lax.rsqrt(deg_ref[...])
    h = jnp.dot(fts_ref[...], w_ref[...],
                preferred_element_type=jnp.float32) * dinv
    out_ref[:n] = h


def _mid_body(n, y_ref, deg_ref, b_ref, w_ref, out_ref):
    dinv = lax.rsqrt(deg_ref[...])
    ysum = (y_ref[0, :n] + y_ref[1, :n]) * dinv
    x = jnp.maximum(ysum + b_ref[...], 0.0)
    out_ref[:n] = jnp.dot(x, w_ref[...],
                          preferred_element_type=jnp.float32) * dinv


def _final_body(n, y_ref, deg_ref, b_ref, wc_ref, bc_ref, out_ref, hid_ref):
    dinv = lax.rsqrt(deg_ref[...])
    ysum = (y_ref[0, :n] + y_ref[1, :n]) * dinv
    x = jnp.maximum(ysum + b_ref[...], 0.0)
    hid_ref[...] = x
    out_ref[...] = jnp.dot(x, wc_ref[...],
                           preferred_element_type=jnp.float32) + bc_ref[...]


def kernel(fts, edge_index, W1, b1, W2, b2, Wc, bc):
    n, in_dim = fts.shape
    hid_dim = W1.shape[1]
    out_dim = Wc.shape[1]
    e = edge_index.shape[1]

    n_pad = ((n + 1 + NS * 8 - 1) // (NS * 8)) * (NS * 8)
    e_quant = NC * NS * CH * IDX_Q
    e_pad = ((e + e_quant - 1) // e_quant) * e_quant

    src = edge_index[0]
    dst = edge_index[1]
    pad = e_pad - e
    src_p = jnp.concatenate([src, jnp.zeros((pad,), jnp.int32)])
    dst_p = jnp.concatenate([dst, jnp.full((pad,), n, jnp.int32)])
    idx3 = jnp.stack([src_p.reshape(-1, CH), dst_p.reshape(-1, CH)], axis=1)
    ones_pad = jnp.ones((n_pad,), jnp.float32)
    zeros_rows = jnp.zeros((n_pad, hid_dim), jnp.float32)

    deg_kernel = _make_deg_kernel(n_pad, e_pad)
    agg_kernel = _make_agg_kernel(n_pad, e_pad, hid_dim)

    deg_full = deg_kernel(dst_p, ones_pad)
    deg = deg_full[:n].reshape(n, 1)

    b1r = b1.reshape(1, hid_dim)
    b2r = b2.reshape(1, hid_dim)
    bcr = bc.reshape(1, out_dim)

    h1 = pl.pallas_call(
        functools.partial(_mm_scale_body, n),
        out_shape=jax.ShapeDtypeStruct((n_pad, hid_dim), jnp.float32),
    )(fts, W1, deg)

    y1 = agg_kernel(h1, zeros_rows, idx3)

    h2 = pl.pallas_call(
        functools.partial(_mid_body, n),
        out_shape=jax.ShapeDtypeStruct((n_pad, hid_dim), jnp.float32),
    )(y1, deg, b1r, W2)

    y2 = agg_kernel(h2, zeros_rows, idx3)

    out, hid = pl.pallas_call(
        functools.partial(_final_body, n),
        out_shape=(
            jax.ShapeDtypeStruct((n, out_dim), jnp.float32),
            jax.ShapeDtypeStruct((n, hid_dim), jnp.float32),
        ),
    )(y2, deg, b2r, Wc, bcr)

    return (out, hid)


# speedup vs baseline: 1.2164x; 1.2164x over previous
import functools

import jax
import jax.numpy as jnp
from jax import lax
from jax.experimental import pallas as pl
from jax.experimental.pallas import tpu as pltpu
from jax.experimental.pallas import tpu_sc as plsc

NC = 2
NS = 16
CH = 128


def _sc_mesh():
    return plsc.VectorSubcoreMesh(core_axis_name="c", subcore_axis_name="s")


def _make_deg_kernel(n_pad, e_pad):
    chunks_per_tile = e_pad // (NC * NS * CH)
    rows_per_tile = n_pad // NS

    @functools.partial(
        pl.kernel,
        mesh=_sc_mesh(),
        out_type=jax.ShapeDtypeStruct((NC * n_pad,), jnp.float32),
        scratch_types=[
            pltpu.VMEM((CH,), jnp.int32),
            pltpu.VMEM((CH,), jnp.float32),
            pltpu.VMEM((rows_per_tile,), jnp.float32),
            pltpu.VMEM_SHARED((n_pad,), jnp.float32),
            pltpu.SemaphoreType.DMA,
        ],
    )
    def deg_kernel(dst_hbm, ones_hbm, zeros_hbm, out_hbm,
                   dst_v, ones_v, row_v, deg_sh, sem):
        c = lax.axis_index("c")
        s = lax.axis_index("s")
        row0 = s * rows_per_tile

        @pl.when(c == 0)
        def _():
            pltpu.sync_copy(ones_hbm.at[pl.ds(row0, rows_per_tile)], row_v)

        @pl.when(c != 0)
        def _():
            pltpu.sync_copy(zeros_hbm.at[pl.ds(row0, rows_per_tile)], row_v)

        pltpu.sync_copy(row_v, deg_sh.at[pl.ds(row0, rows_per_tile)])
        pltpu.sync_copy(ones_hbm.at[pl.ds(0, CH)], ones_v)
        plsc.subcore_barrier()

        base_chunk = (c * NS + s) * chunks_per_tile

        @pl.loop(0, chunks_per_tile)
        def _(j):
            off = (base_chunk + j) * CH
            pltpu.sync_copy(dst_hbm.at[pl.ds(off, CH)], dst_v)
            pltpu.sync_copy(ones_v, deg_sh.at[dst_v], add=True)

        plsc.subcore_barrier()

        pltpu.sync_copy(deg_sh.at[pl.ds(row0, rows_per_tile)], row_v)
        pltpu.sync_copy(row_v, out_hbm.at[pl.ds(c * n_pad + row0, rows_per_tile)])

    return deg_kernel


IDX_Q = 8


def _make_agg_kernel(n_pad, e_pad, dim):
    chunks_per_tile = e_pad // (NC * NS * CH)
    assert chunks_per_tile % IDX_Q == 0
    rows_per_tile = n_pad // NS
    n_init_chunks = (rows_per_tile + CH - 1) // CH

    @functools.partial(
        pl.kernel,
        mesh=_sc_mesh(),
        out_type=jax.ShapeDtypeStruct((NC, n_pad, dim), jnp.float32),
        scratch_types=[
            pltpu.VMEM((CH,), jnp.int32),
            pltpu.VMEM((CH,), jnp.int32),
            pltpu.VMEM((CH, dim), jnp.float32),
            pltpu.VMEM_SHARED((n_pad, dim), jnp.float32),
            pltpu.SemaphoreType.DMA,
        ],
    )
    def agg_kernel(h_hbm, zeros_hbm, src_hbm, dst_hbm, out_hbm,
                   src_v, dst_v, rows_v, y_sh, sem):
        c = lax.axis_index("c")
        s = lax.axis_index("s")
        row0 = s * rows_per_tile

        for k in range(n_init_chunks):
            r = row0 + k * CH
            m = min(CH, rows_per_tile - k * CH)

            @pl.when(c == 0)
            def _():
                pltpu.sync_copy(h_hbm.at[pl.ds(r, m)],
                                rows_v.at[pl.ds(0, m)])

            @pl.when(c != 0)
            def _():
                pltpu.sync_copy(zeros_hbm.at[pl.ds(r, m)],
                                rows_v.at[pl.ds(0, m)])

            pltpu.sync_copy(rows_v.at[pl.ds(0, m)], y_sh.at[pl.ds(r, m)])

        plsc.subcore_barrier()

        base_chunk = (c * NS + s) * chunks_per_tile

        @pl.loop(0, chunks_per_tile)
        def _(j):
            off = (base_chunk + j) * CH
            pltpu.sync_copy(src_hbm.at[pl.ds(off, CH)], src_v)
            pltpu.sync_copy(dst_hbm.at[pl.ds(off, CH)], dst_v)
            pltpu.async_copy(h_hbm.at[src_v], rows_v, sem).wait()
            pltpu.sync_copy(rows_v, y_sh.at[dst_v], add=True)

        plsc.subcore_barrier()

        for k in range(n_init_chunks):
            r = row0 + k * CH
            m = min(CH, rows_per_tile - k * CH)
            pltpu.sync_copy(y_sh.at[pl.ds(r, m)], rows_v.at[pl.ds(0, m)])
            pltpu.sync_copy(rows_v.at[pl.ds(0, m)],
                            out_hbm.at[c, pl.ds(r, m), :])

    return agg_kernel


def _mm_scale_body(n, fts_ref, w_ref, deg_ref, out_ref):
    dinv = lax.rsqrt(deg_ref[...])
    h = jnp.dot(fts_ref[...], w_ref[...],
                preferred_element_type=jnp.float32) * dinv
    out_ref[:n] = h


def _mid_body(n, y_ref, deg_ref, b_ref, w_ref, out_ref):
    dinv = lax.rsqrt(deg_ref[...])
    ysum = (y_ref[0, :n] + y_ref[1, :n]) * dinv
    x = jnp.maximum(ysum + b_ref[...], 0.0)
    out_ref[:n] = jnp.dot(x, w_ref[...],
                          preferred_element_type=jnp.float32) * dinv


def _final_body(n, y_ref, deg_ref, b_ref, wc_ref, bc_ref, out_ref, hid_ref):
    dinv = lax.rsqrt(deg_ref[...])
    ysum = (y_ref[0, :n] + y_ref[1, :n]) * dinv
    x = jnp.maximum(ysum + b_ref[...], 0.0)
    hid_ref[...] = x
    out_ref[...] = jnp.dot(x, wc_ref[...],
                           preferred_element_type=jnp.float32) + bc_ref[...]


def kernel(fts, edge_index, W1, b1, W2, b2, Wc, bc):
    n, in_dim = fts.shape
    hid_dim = W1.shape[1]
    out_dim = Wc.shape[1]
    e = edge_index.shape[1]

    n_pad = ((n + 1 + NS * 8 - 1) // (NS * 8)) * (NS * 8)
    e_quant = NC * NS * CH * IDX_Q
    e_pad = ((e + e_quant - 1) // e_quant) * e_quant

    src = edge_index[0]
    dst = edge_index[1]
    pad = e_pad - e
    src_p = jnp.concatenate([src, jnp.zeros((pad,), jnp.int32)])
    dst_p = jnp.concatenate([dst, jnp.full((pad,), n, jnp.int32)])
    ones_pad = jnp.ones((n_pad,), jnp.float32)
    zeros_rows = jnp.zeros((n_pad, hid_dim), jnp.float32)

    deg_kernel = _make_deg_kernel(n_pad, e_pad)
    agg_kernel = _make_agg_kernel(n_pad, e_pad, hid_dim)

    zeros_1d = jnp.zeros((n_pad,), jnp.float32)
    deg_full = deg_kernel(dst_p, ones_pad, zeros_1d)
    deg = (deg_full[:n] + deg_full[n_pad:n_pad + n]).reshape(n, 1)

    b1r = b1.reshape(1, hid_dim)
    b2r = b2.reshape(1, hid_dim)
    bcr = bc.reshape(1, out_dim)

    h1 = pl.pallas_call(
        functools.partial(_mm_scale_body, n),
        out_shape=jax.ShapeDtypeStruct((n_pad, hid_dim), jnp.float32),
    )(fts, W1, deg)

    y1 = agg_kernel(h1, zeros_rows, src_p, dst_p)

    h2 = pl.pallas_call(
        functools.partial(_mid_body, n),
        out_shape=jax.ShapeDtypeStruct((n_pad, hid_dim), jnp.float32),
    )(y1, deg, b1r, W2)

    y2 = agg_kernel(h2, zeros_rows, src_p, dst_p)

    out, hid = pl.pallas_call(
        functools.partial(_final_body, n),
        out_shape=(
            jax.ShapeDtypeStruct((n, out_dim), jnp.float32),
            jax.ShapeDtypeStruct((n, hid_dim), jnp.float32),
        ),
    )(y2, deg, b2r, Wc, bcr)

    return (out, hid)


# speedup vs baseline: 1.6389x; 1.3474x over previous
import functools

import jax
import jax.numpy as jnp
from jax import lax
from jax.experimental import pallas as pl
from jax.experimental.pallas import tpu as pltpu
from jax.experimental.pallas import tpu_sc as plsc

NC = 2
NS = 16
CH = 128


def _sc_mesh():
    return plsc.VectorSubcoreMesh(core_axis_name="c", subcore_axis_name="s")


def _make_deg_kernel(n_pad, e_pad):
    chunks_per_tile = e_pad // (NC * NS * CH)
    rows_per_tile = n_pad // NS

    @functools.partial(
        pl.kernel,
        mesh=_sc_mesh(),
        out_type=jax.ShapeDtypeStruct((NC * n_pad,), jnp.float32),
        scratch_types=[
            pltpu.VMEM((CH,), jnp.int32),
            pltpu.VMEM((CH,), jnp.float32),
            pltpu.VMEM((rows_per_tile,), jnp.float32),
            pltpu.VMEM_SHARED((n_pad,), jnp.float32),
            pltpu.SemaphoreType.DMA,
        ],
    )
    def deg_kernel(dst_hbm, ones_hbm, zeros_hbm, out_hbm,
                   dst_v, ones_v, row_v, deg_sh, sem):
        c = lax.axis_index("c")
        s = lax.axis_index("s")
        row0 = s * rows_per_tile

        @pl.when(c == 0)
        def _():
            pltpu.sync_copy(ones_hbm.at[pl.ds(row0, rows_per_tile)], row_v)

        @pl.when(c != 0)
        def _():
            pltpu.sync_copy(zeros_hbm.at[pl.ds(row0, rows_per_tile)], row_v)

        pltpu.sync_copy(row_v, deg_sh.at[pl.ds(row0, rows_per_tile)])
        pltpu.sync_copy(ones_hbm.at[pl.ds(0, CH)], ones_v)
        plsc.subcore_barrier()

        base_chunk = (c * NS + s) * chunks_per_tile

        @pl.loop(0, chunks_per_tile)
        def _(j):
            off = (base_chunk + j) * CH
            pltpu.sync_copy(dst_hbm.at[pl.ds(off, CH)], dst_v)
            pltpu.sync_copy(ones_v, deg_sh.at[dst_v], add=True)

        plsc.subcore_barrier()

        pltpu.sync_copy(deg_sh.at[pl.ds(row0, rows_per_tile)], row_v)
        pltpu.sync_copy(row_v, out_hbm.at[pl.ds(c * n_pad + row0, rows_per_tile)])

    return deg_kernel


IDX_Q = 8


def _make_agg_kernel(n_pad, e_pad, dim):
    chunks_per_tile = e_pad // (NC * NS * CH)
    rows_per_tile = n_pad // NS
    n_init_chunks = (rows_per_tile + CH - 1) // CH

    @functools.partial(
        pl.kernel,
        mesh=_sc_mesh(),
        out_type=jax.ShapeDtypeStruct((NC, n_pad, dim), jnp.float32),
        scratch_types=[
            pltpu.VMEM((CH,), jnp.int32),
            pltpu.VMEM((CH,), jnp.int32),
            pltpu.VMEM((CH, dim), jnp.float32),
            pltpu.VMEM_SHARED((n_pad, dim), jnp.float32),
            pltpu.SemaphoreType.DMA,
        ],
    )
    def agg_kernel(h_hbm, zeros_hbm, src_hbm, dst_hbm, out_hbm,
                   src_v, dst_v, rows_v, y_sh, sem):
        c = lax.axis_index("c")
        s = lax.axis_index("s")
        row0 = s * rows_per_tile

        for k in range(n_init_chunks):
            r = row0 + k * CH
            m = min(CH, rows_per_tile - k * CH)

            @pl.when(c == 0)
            def _():
                pltpu.sync_copy(h_hbm.at[pl.ds(r, m)],
                                rows_v.at[pl.ds(0, m)])

            @pl.when(c != 0)
            def _():
                pltpu.sync_copy(zeros_hbm.at[pl.ds(r, m)],
                                rows_v.at[pl.ds(0, m)])

            pltpu.sync_copy(rows_v.at[pl.ds(0, m)], y_sh.at[pl.ds(r, m)])

        plsc.subcore_barrier()

        base_chunk = (c * NS + s) * chunks_per_tile

        @pl.loop(0, chunks_per_tile)
        def _(j):
            off = (base_chunk + j) * CH
            pltpu.sync_copy(src_hbm.at[pl.ds(off, CH)], src_v)
            pltpu.sync_copy(dst_hbm.at[pl.ds(off, CH)], dst_v)
            pltpu.async_copy(h_hbm.at[src_v], rows_v, sem).wait()
            pltpu.sync_copy(rows_v, y_sh.at[dst_v], add=True)

        plsc.subcore_barrier()

        for k in range(n_init_chunks):
            r = row0 + k * CH
            m = min(CH, rows_per_tile - k * CH)
            pltpu.sync_copy(y_sh.at[pl.ds(r, m)], rows_v.at[pl.ds(0, m)])
            pltpu.sync_copy(rows_v.at[pl.ds(0, m)],
                            out_hbm.at[c, pl.ds(r, m), :])

    return agg_kernel


def _mm_scale_body(n, fts_ref, w_ref, deg_ref, out_ref):
    dinv = lax.rsqrt(deg_ref[...])
    h = jnp.dot(fts_ref[...], w_ref[...],
                preferred_element_type=jnp.float32) * dinv
    out_ref[:n] = h


def _mid_body(n, y_ref, deg_ref, b_ref, w_ref, out_ref):
    dinv = lax.rsqrt(deg_ref[...])
    ysum = (y_ref[0, :n] + y_ref[1, :n]) * dinv
    x = jnp.maximum(ysum + b_ref[...], 0.0)
    out_ref[:n] = jnp.dot(x, w_ref[...],
                          preferred_element_type=jnp.float32) * dinv


def _final_body(n, y_ref, deg_ref, b_ref, wc_ref, bc_ref, out_ref, hid_ref):
    dinv = lax.rsqrt(deg_ref[...])
    ysum = (y_ref[0, :n] + y_ref[1, :n]) * dinv
    x = jnp.maximum(ysum + b_ref[...], 0.0)
    hid_ref[...] = x
    out_ref[...] = jnp.dot(x, wc_ref[...],
                           preferred_element_type=jnp.float32) + bc_ref[...]


def kernel(fts, edge_index, W1, b1, W2, b2, Wc, bc):
    n, in_dim = fts.shape
    hid_dim = W1.shape[1]
    out_dim = Wc.shape[1]
    e = edge_index.shape[1]

    n_pad = ((n + 1 + NS * 8 - 1) // (NS * 8)) * (NS * 8)
    e_quant = NC * NS * CH
    e_pad = ((e + e_quant - 1) // e_quant) * e_quant

    src = edge_index[0]
    dst = edge_index[1]
    pad = e_pad - e
    src_p = jnp.concatenate([src, jnp.zeros((pad,), jnp.int32)])
    pad_dst = n + jnp.arange(pad, dtype=jnp.int32) % (n_pad - n)
    dst_p = jnp.concatenate([dst, pad_dst])
    ones_pad = jnp.ones((n_pad,), jnp.float32)
    zeros_rows = jnp.zeros((n_pad, hid_dim), jnp.float32)

    deg_kernel = _make_deg_kernel(n_pad, e_pad)
    agg_kernel = _make_agg_kernel(n_pad, e_pad, hid_dim)

    zeros_1d = jnp.zeros((n_pad,), jnp.float32)
    deg_full = deg_kernel(dst_p, ones_pad, zeros_1d)
    deg = (deg_full[:n] + deg_full[n_pad:n_pad + n]).reshape(n, 1)

    b1r = b1.reshape(1, hid_dim)
    b2r = b2.reshape(1, hid_dim)
    bcr = bc.reshape(1, out_dim)

    h1 = pl.pallas_call(
        functools.partial(_mm_scale_body, n),
        out_shape=jax.ShapeDtypeStruct((n_pad, hid_dim), jnp.float32),
    )(fts, W1, deg)

    y1 = agg_kernel(h1, zeros_rows, src_p, dst_p)

    h2 = pl.pallas_call(
        functools.partial(_mid_body, n),
        out_shape=jax.ShapeDtypeStruct((n_pad, hid_dim), jnp.float32),
    )(y1, deg, b1r, W2)

    y2 = agg_kernel(h2, zeros_rows, src_p, dst_p)

    out, hid = pl.pallas_call(
        functools.partial(_final_body, n),
        out_shape=(
            jax.ShapeDtypeStruct((n, out_dim), jnp.float32),
            jax.ShapeDtypeStruct((n, hid_dim), jnp.float32),
        ),
    )(y2, deg, b2r, Wc, bcr)

    return (out, hid)
